# baseline (speedup 1.0000x reference)
import functools

import jax
import jax.numpy as jnp
from jax import lax
from jax.experimental import pallas as pl
from jax.experimental.pallas import tpu as pltpu
from jax.experimental.pallas import tpu_sc as plsc

E = 8
D = 1024
FF = 2048
S = 2048
LANES = 128
T = 256
NT = 24
NP = NT * T
NB = S // LANES
NW = 32
TPW = S // NW


def _plan_body(x_ref, gw_ref, eb_ref, d0_ref, d1_ref, w0_ref, w1_ref,
               eid_ref):
    f32 = jnp.float32
    logits = lax.dot_general(x_ref[...], gw_ref[...], (((1,), (1,)), ((), ())),
                             preferred_element_type=f32)
    scores = jax.nn.sigmoid(logits)
    lane = lax.broadcasted_iota(jnp.int32, (S, LANES), 1)
    valid = lane < E
    neg = f32(-1e30)
    swb = jnp.where(valid, scores + eb_ref[...], neg)
    m1 = jnp.max(swb, axis=1, keepdims=True)
    i1 = jnp.min(jnp.where(swb == m1, lane, LANES), axis=1, keepdims=True)
    sel1 = lane == i1
    s1 = jnp.sum(jnp.where(sel1, scores, 0.0), axis=1, keepdims=True)
    swb2 = jnp.where(sel1, neg, swb)
    m2 = jnp.max(swb2, axis=1, keepdims=True)
    i2 = jnp.min(jnp.where(swb2 == m2, lane, LANES), axis=1, keepdims=True)
    sel2 = lane == i2
    s2 = jnp.sum(jnp.where(sel2, scores, 0.0), axis=1, keepdims=True)
    denom = s1 + s2 + 1e-20
    w0_ref[...] = jnp.broadcast_to(s1 / denom, (S, LANES))
    w1_ref[...] = jnp.broadcast_to(s2 / denom, (S, LANES))

    oh1 = sel1.astype(f32)
    oh2 = sel2.astype(f32)
    cnt1 = jnp.sum(oh1, axis=0, keepdims=True)
    cnt2 = jnp.sum(oh2, axis=0, keepdims=True)
    cnt = cnt1 + cnt2
    ntile = jnp.floor((cnt + (T - 1)) * (1.0 / T))
    r_ii = lax.broadcasted_iota(jnp.int32, (LANES, LANES), 0)
    c_ii = lax.broadcasted_iota(jnp.int32, (LANES, LANES), 1)
    lower = (r_ii > c_ii).astype(f32)
    upper = (r_ii < c_ii).astype(f32)
    diag = r_ii == c_ii
    off = T * lax.dot_general(ntile, upper, (((1,), (0,)), ((), ())),
                              preferred_element_type=f32)

    run1 = jnp.zeros((1, LANES), f32)
    run2 = jnp.zeros((1, LANES), f32)
    for b in range(NB):
        sl = slice(b * LANES, (b + 1) * LANES)
        a1 = oh1[sl]
        a2 = oh2[sl]
        rank1 = lax.dot_general(lower, a1, (((1,), (0,)), ((), ())),
                                preferred_element_type=f32) + run1
        rank2 = lax.dot_general(lower, a2, (((1,), (0,)), ((), ())),
                                preferred_element_type=f32) + run2 + cnt1
        run1 = run1 + jnp.sum(a1, axis=0, keepdims=True)
        run2 = run2 + jnp.sum(a2, axis=0, keepdims=True)
        d0b = jnp.sum(jnp.where(sel1[sl], off + rank1, 0.0), axis=1,
                      keepdims=True)
        d1b = jnp.sum(jnp.where(sel2[sl], off + rank2, 0.0), axis=1,
                      keepdims=True)
        d0_ref[b:b + 1, :] = jnp.sum(
            jnp.where(diag, jnp.broadcast_to(d0b, (LANES, LANES)), 0.0),
            axis=0, keepdims=True).astype(jnp.int32)
        d1_ref[b:b + 1, :] = jnp.sum(
            jnp.where(diag, jnp.broadcast_to(d1b, (LANES, LANES)), 0.0),
            axis=0, keepdims=True).astype(jnp.int32)

    offb = jnp.broadcast_to(off, (LANES, LANES))
    off_col = jnp.sum(jnp.where(diag, offb, 0.0), axis=1,
                      keepdims=True)
    ge = ((off_col <= (c_ii * T).astype(f32)) & (r_ii < E)).astype(jnp.int32)
    eid_ref[...] = jnp.sum(ge, axis=0, keepdims=True) - 1


def _ffn_body(meta_ref, xs_ref, wg_any, wu_any, wd_any, ws_ref, ys_ref,
              wgb, wub, wdb, wgc, wuc, wdc, sems):
    t = pl.program_id(0)
    par = meta_ref[0, t]
    first = meta_ref[1, t]
    nxte = meta_ref[2, t]
    pref = meta_ref[3, t]
    ecur = meta_ref[4, t]

    def _copies(e):
        return (
            pltpu.make_async_copy(wg_any.at[e], wgb, sems),
            pltpu.make_async_copy(wu_any.at[e], wub, sems),
            pltpu.make_async_copy(wd_any.at[e], wdb, sems),
        )

    @pl.when(t == 0)
    def _prime():
        for c in _copies(ecur):
            c.start()

    @pl.when(first == 1)
    def _wait():
        for c in _copies(ecur):
            c.wait()
        wgc[par] = wgb[...].astype(jnp.bfloat16)
        wuc[par] = wub[...].astype(jnp.bfloat16)
        wdc[par] = wdb[...].astype(jnp.bfloat16)

    @pl.when(pref == 1)
    def _prefetch():
        for c in _copies(nxte):
            c.start()

    xbf = xs_ref[...].astype(jnp.bfloat16)
    g = lax.dot_general(xbf, wgc[par], (((1,), (1,)), ((), ())),
                        preferred_element_type=jnp.float32)
    u = lax.dot_general(xbf, wuc[par], (((1,), (1,)), ((), ())),
                        preferred_element_type=jnp.float32)
    h = (g * jax.nn.sigmoid(g) * u).astype(jnp.bfloat16)
    y = lax.dot_general(h, wdc[par], (((1,), (1,)), ((), ())),
                        preferred_element_type=jnp.float32)
    ys_ref[...] = y * ws_ref[:, :1]


def _plan(x2, gwp, ebp):
    return pl.pallas_call(
        _plan_body,
        out_shape=(
            jax.ShapeDtypeStruct((NB, LANES), jnp.int32),
            jax.ShapeDtypeStruct((NB, LANES), jnp.int32),
            jax.ShapeDtypeStruct((S, LANES), jnp.float32),
            jax.ShapeDtypeStruct((S, LANES), jnp.float32),
            jax.ShapeDtypeStruct((1, LANES), jnp.int32),
        ),
        compiler_params=pltpu.CompilerParams(vmem_limit_bytes=100 * 1024 * 1024),
    )(x2, gwp, ebp)


def _ffn(meta, xs, Wg, Wu, Wd, wslot):
    grid_spec = pltpu.PrefetchScalarGridSpec(
        num_scalar_prefetch=1,
        grid=(NT,),
        in_specs=[
            pl.BlockSpec((T, D), lambda t, m: (t, 0)),
            pl.BlockSpec(memory_space=pl.ANY),
            pl.BlockSpec(memory_space=pl.ANY),
            pl.BlockSpec(memory_space=pl.ANY),
            pl.BlockSpec((T, LANES), lambda t, m: (t, 0)),
        ],
        out_specs=pl.BlockSpec((T, D), lambda t, m: (t, 0)),
        scratch_shapes=[
            pltpu.VMEM((FF, D), jnp.float32),
            pltpu.VMEM((FF, D), jnp.float32),
            pltpu.VMEM((D, FF), jnp.float32),
            pltpu.VMEM((2, FF, D), jnp.bfloat16),
            pltpu.VMEM((2, FF, D), jnp.bfloat16),
            pltpu.VMEM((2, D, FF), jnp.bfloat16),
            pltpu.SemaphoreType.DMA,
        ],
    )
    return pl.pallas_call(
        _ffn_body,
        grid_spec=grid_spec,
        out_shape=jax.ShapeDtypeStruct((NP, D), jnp.float32),
        compiler_params=pltpu.CompilerParams(
            dimension_semantics=("arbitrary",),
            vmem_limit_bytes=100 * 1024 * 1024,
        ),
    )(meta, xs, Wg, Wu, Wd, wslot)


def _dispatch(x2, d0, d1, w0f, w1f):
    mesh = plsc.VectorSubcoreMesh(core_axis_name="c", subcore_axis_name="s")

    @functools.partial(
        pl.kernel, mesh=mesh,
        out_type=(
            jax.ShapeDtypeStruct((NP, D), jnp.float32),
            jax.ShapeDtypeStruct((NP, LANES), jnp.float32),
        ),
        scratch_types=[
            pltpu.VMEM((TPW, D), jnp.float32),
            pltpu.VMEM((TPW, LANES), jnp.float32),
            pltpu.VMEM((TPW, LANES), jnp.float32),
            pltpu.VMEM((TPW,), jnp.int32),
            pltpu.VMEM((TPW,), jnp.int32),
            pltpu.SemaphoreType.DMA,
        ],
    )
    def scatter(x_hbm, d0_hbm, d1_hbm, w0_hbm, w1_hbm, xs_hbm, ws_hbm,
                xbuf, wbuf0, wbuf1, idx0, idx1, sem):
        wid = lax.axis_index("s") * 2 + lax.axis_index("c")
        base = wid * TPW
        pltpu.sync_copy(x_hbm.at[pl.ds(base, TPW)], xbuf)
        pltpu.sync_copy(d0_hbm.at[pl.ds(base, TPW)], idx0)
        pltpu.sync_copy(d1_hbm.at[pl.ds(base, TPW)], idx1)
        pltpu.sync_copy(w0_hbm.at[pl.ds(base, TPW)], wbuf0)
        pltpu.sync_copy(w1_hbm.at[pl.ds(base, TPW)], wbuf1)
        c1 = pltpu.async_copy(xbuf, xs_hbm.at[idx0], sem)
        c2 = pltpu.async_copy(xbuf, xs_hbm.at[idx1], sem)
        c3 = pltpu.async_copy(wbuf0, ws_hbm.at[idx0], sem)
        c4 = pltpu.async_copy(wbuf1, ws_hbm.at[idx1], sem)
        c1.wait()
        c2.wait()
        c3.wait()
        c4.wait()

    return scatter(x2, d0, d1, w0f, w1f)


def _combine(ys, d0, d1):
    mesh = plsc.VectorSubcoreMesh(core_axis_name="c", subcore_axis_name="s")
    HC = TPW // 2

    @functools.partial(
        pl.kernel, mesh=mesh,
        out_type=jax.ShapeDtypeStruct((S, D), jnp.float32),
        scratch_types=[
            pltpu.VMEM((HC, D), jnp.float32),
            pltpu.VMEM((HC, D), jnp.float32),
            pltpu.VMEM((HC,), jnp.int32),
            pltpu.VMEM((HC,), jnp.int32),
            pltpu.SemaphoreType.DMA,
        ],
    )
    def gather_add(ys_hbm, d0_hbm, d1_hbm, out_hbm, buf0, buf1, idx0, idx1,
                   sem):
        wid = lax.axis_index("s") * 2 + lax.axis_index("c")
        for half in range(2):
            base = wid * TPW + half * HC
            pltpu.sync_copy(d0_hbm.at[pl.ds(base, HC)], idx0)
            pltpu.sync_copy(d1_hbm.at[pl.ds(base, HC)], idx1)
            c1 = pltpu.async_copy(ys_hbm.at[idx0], buf0, sem)
            c2 = pltpu.async_copy(ys_hbm.at[idx1], buf1, sem)
            c1.wait()
            c2.wait()

            def row_add(r, carry):
                for cc in range(D // 16):
                    v = buf0[r, pl.ds(cc * 16, 16)] + buf1[r, pl.ds(cc * 16, 16)]
                    buf0[r, pl.ds(cc * 16, 16)] = v
                return carry

            lax.fori_loop(0, HC, row_add, 0)
            pltpu.sync_copy(buf0, out_hbm.at[pl.ds(base, HC)])

    return gather_add(ys, d0, d1)


@jax.jit
def kernel(x, gate_w, e_bias, Wg, Wu, Wd):
    b, s, d = x.shape
    x2 = x.reshape(s, d)
    gwp = jnp.zeros((LANES, D), jnp.float32).at[:E].set(gate_w)
    ebp = jnp.zeros((1, LANES), jnp.float32).at[0, :E].set(e_bias)

    d0s, d1s, w0f, w1f, eidr = _plan(x2, gwp, ebp)
    d0 = d0s.reshape(S)
    d1 = d1s.reshape(S)
    eid = eidr[0, :NT]

    idxs = jnp.arange(NT, dtype=jnp.int32)
    prev = jnp.concatenate([jnp.full((1,), -1, jnp.int32), eid[:-1]])
    first = (eid != prev).astype(jnp.int32)
    par = (jnp.cumsum(first) - 1) & 1
    fidx = jnp.where(first == 1, idxs, NT + 1)
    nxt = lax.cummin(fidx[::-1])[::-1]
    nxt = jnp.concatenate([nxt[1:], jnp.full((1,), NT + 1, jnp.int32)])
    pref = ((first == 1) & (nxt < NT)).astype(jnp.int32)
    nxte = eid[jnp.clip(nxt, 0, NT - 1)]
    meta = jnp.stack([par, first, nxte, pref, eid]).astype(jnp.int32)

    xs, wslot = _dispatch(x2, d0, d1, w0f, w1f)
    ys = _ffn(meta, xs, Wg, Wu, Wd, wslot)
    out = _combine(ys, d0, d1)
    return out.reshape(b, s, d)


# speedup vs baseline: 1.1665x; 1.1665x over previous
import functools

import jax
import jax.numpy as jnp
from jax import lax
from jax.experimental import pallas as pl
from jax.experimental.pallas import tpu as pltpu
from jax.experimental.pallas import tpu_sc as plsc

E = 8
D = 1024
FF = 2048
S = 2048
LANES = 128
T = 256
NT = 24
NP = NT * T
NB = S // LANES
NW = 32
TPW = S // NW


def _plan_body(x_ref, gw_ref, eb_ref, d0_ref, d1_ref, w0_ref, w1_ref,
               eid_ref):
    f32 = jnp.float32
    logits = lax.dot_general(x_ref[...], gw_ref[...], (((1,), (1,)), ((), ())),
                             preferred_element_type=f32)
    scores = jax.nn.sigmoid(logits)
    lane = lax.broadcasted_iota(jnp.int32, (S, LANES), 1)
    valid = lane < E
    neg = f32(-1e30)
    swb = jnp.where(valid, scores + eb_ref[...], neg)
    m1 = jnp.max(swb, axis=1, keepdims=True)
    i1 = jnp.min(jnp.where(swb == m1, lane, LANES), axis=1, keepdims=True)
    sel1 = lane == i1
    s1 = jnp.sum(jnp.where(sel1, scores, 0.0), axis=1, keepdims=True)
    swb2 = jnp.where(sel1, neg, swb)
    m2 = jnp.max(swb2, axis=1, keepdims=True)
    i2 = jnp.min(jnp.where(swb2 == m2, lane, LANES), axis=1, keepdims=True)
    sel2 = lane == i2
    s2 = jnp.sum(jnp.where(sel2, scores, 0.0), axis=1, keepdims=True)
    denom = s1 + s2 + 1e-20
    w0_ref[...] = jnp.broadcast_to(s1 / denom, (S, LANES))
    w1_ref[...] = jnp.broadcast_to(s2 / denom, (S, LANES))

    oh1 = sel1.astype(f32)
    oh2 = sel2.astype(f32)
    cnt1 = jnp.sum(oh1, axis=0, keepdims=True)
    cnt2 = jnp.sum(oh2, axis=0, keepdims=True)
    cnt = cnt1 + cnt2
    ntile = jnp.floor((cnt + (T - 1)) * (1.0 / T))
    r_ii = lax.broadcasted_iota(jnp.int32, (LANES, LANES), 0)
    c_ii = lax.broadcasted_iota(jnp.int32, (LANES, LANES), 1)
    lower = (r_ii > c_ii).astype(f32)
    upper = (r_ii < c_ii).astype(f32)
    diag = r_ii == c_ii
    off = T * lax.dot_general(ntile, upper, (((1,), (0,)), ((), ())),
                              preferred_element_type=f32)

    run1 = jnp.zeros((1, LANES), f32)
    run2 = jnp.zeros((1, LANES), f32)
    for b in range(NB):
        sl = slice(b * LANES, (b + 1) * LANES)
        a1 = oh1[sl]
        a2 = oh2[sl]
        rank1 = lax.dot_general(lower, a1, (((1,), (0,)), ((), ())),
                                preferred_element_type=f32) + run1
        rank2 = lax.dot_general(lower, a2, (((1,), (0,)), ((), ())),
                                preferred_element_type=f32) + run2 + cnt1
        run1 = run1 + jnp.sum(a1, axis=0, keepdims=True)
        run2 = run2 + jnp.sum(a2, axis=0, keepdims=True)
        d0b = jnp.sum(jnp.where(sel1[sl], off + rank1, 0.0), axis=1,
                      keepdims=True)
        d1b = jnp.sum(jnp.where(sel2[sl], off + rank2, 0.0), axis=1,
                      keepdims=True)
        d0_ref[b:b + 1, :] = jnp.sum(
            jnp.where(diag, jnp.broadcast_to(d0b, (LANES, LANES)), 0.0),
            axis=0, keepdims=True).astype(jnp.int32)
        d1_ref[b:b + 1, :] = jnp.sum(
            jnp.where(diag, jnp.broadcast_to(d1b, (LANES, LANES)), 0.0),
            axis=0, keepdims=True).astype(jnp.int32)

    offb = jnp.broadcast_to(off, (LANES, LANES))
    off_col = jnp.sum(jnp.where(diag, offb, 0.0), axis=1,
                      keepdims=True)
    ge = ((off_col <= (c_ii * T).astype(f32)) & (r_ii < E)).astype(jnp.int32)
    eid_ref[...] = jnp.sum(ge, axis=0, keepdims=True) - 1


def _ffn_body(meta_ref, xs_ref, wg_any, wu_any, wd_any, ws_ref, ys_ref,
              wgb, wub, wdb, sems):
    t = pl.program_id(0)
    par = meta_ref[0, t]
    first = meta_ref[1, t]
    nxte = meta_ref[2, t]
    pref = meta_ref[3, t]
    ecur = meta_ref[4, t]

    def _copies(e, slot):
        return (
            pltpu.make_async_copy(wg_any.at[e], wgb.at[slot], sems.at[slot]),
            pltpu.make_async_copy(wu_any.at[e], wub.at[slot], sems.at[slot]),
            pltpu.make_async_copy(wd_any.at[e], wdb.at[slot], sems.at[slot]),
        )

    @pl.when(t == 0)
    def _prime():
        for c in _copies(ecur, par):
            c.start()

    @pl.when(first == 1)
    def _wait():
        for c in _copies(ecur, par):
            c.wait()

    @pl.when(pref == 1)
    def _prefetch():
        for c in _copies(nxte, 1 - par):
            c.start()

    xbf = xs_ref[...].astype(jnp.bfloat16)
    g = lax.dot_general(xbf, wgb[par].astype(jnp.bfloat16),
                        (((1,), (1,)), ((), ())),
                        preferred_element_type=jnp.float32)
    u = lax.dot_general(xbf, wub[par].astype(jnp.bfloat16),
                        (((1,), (1,)), ((), ())),
                        preferred_element_type=jnp.float32)
    h = (g * jax.nn.sigmoid(g) * u).astype(jnp.bfloat16)
    y = lax.dot_general(h, wdb[par].astype(jnp.bfloat16),
                        (((1,), (1,)), ((), ())),
                        preferred_element_type=jnp.float32)
    ys_ref[...] = y * ws_ref[:, :1]


def _plan(x2, gwp, ebp):
    return pl.pallas_call(
        _plan_body,
        out_shape=(
            jax.ShapeDtypeStruct((NB, LANES), jnp.int32),
            jax.ShapeDtypeStruct((NB, LANES), jnp.int32),
            jax.ShapeDtypeStruct((S, LANES), jnp.float32),
            jax.ShapeDtypeStruct((S, LANES), jnp.float32),
            jax.ShapeDtypeStruct((1, LANES), jnp.int32),
        ),
        compiler_params=pltpu.CompilerParams(vmem_limit_bytes=100 * 1024 * 1024),
    )(x2, gwp, ebp)


def _ffn(meta, xs, Wg, Wu, Wd, wslot):
    grid_spec = pltpu.PrefetchScalarGridSpec(
        num_scalar_prefetch=1,
        grid=(NT,),
        in_specs=[
            pl.BlockSpec((T, D), lambda t, m: (t, 0)),
            pl.BlockSpec(memory_space=pl.ANY),
            pl.BlockSpec(memory_space=pl.ANY),
            pl.BlockSpec(memory_space=pl.ANY),
            pl.BlockSpec((T, LANES), lambda t, m: (t, 0)),
        ],
        out_specs=pl.BlockSpec((T, D), lambda t, m: (t, 0)),
        scratch_shapes=[
            pltpu.VMEM((2, FF, D), jnp.float32),
            pltpu.VMEM((2, FF, D), jnp.float32),
            pltpu.VMEM((2, D, FF), jnp.float32),
            pltpu.SemaphoreType.DMA((2,)),
        ],
    )
    return pl.pallas_call(
        _ffn_body,
        grid_spec=grid_spec,
        out_shape=jax.ShapeDtypeStruct((NP, D), jnp.float32),
        compiler_params=pltpu.CompilerParams(
            dimension_semantics=("arbitrary",),
            vmem_limit_bytes=100 * 1024 * 1024,
        ),
    )(meta, xs, Wg, Wu, Wd, wslot)


def _dispatch(x2, d0, d1, w0f, w1f):
    mesh = plsc.VectorSubcoreMesh(core_axis_name="c", subcore_axis_name="s")

    @functools.partial(
        pl.kernel, mesh=mesh,
        out_type=(
            jax.ShapeDtypeStruct((NP, D), jnp.float32),
            jax.ShapeDtypeStruct((NP, LANES), jnp.float32),
        ),
        scratch_types=[
            pltpu.VMEM((TPW, D), jnp.float32),
            pltpu.VMEM((TPW, LANES), jnp.float32),
            pltpu.VMEM((TPW, LANES), jnp.float32),
            pltpu.VMEM((TPW,), jnp.int32),
            pltpu.VMEM((TPW,), jnp.int32),
            pltpu.SemaphoreType.DMA,
        ],
    )
    def scatter(x_hbm, d0_hbm, d1_hbm, w0_hbm, w1_hbm, xs_hbm, ws_hbm,
                xbuf, wbuf0, wbuf1, idx0, idx1, sem):
        wid = lax.axis_index("s") * 2 + lax.axis_index("c")
        base = wid * TPW
        pltpu.sync_copy(x_hbm.at[pl.ds(base, TPW)], xbuf)
        pltpu.sync_copy(d0_hbm.at[pl.ds(base, TPW)], idx0)
        pltpu.sync_copy(d1_hbm.at[pl.ds(base, TPW)], idx1)
        pltpu.sync_copy(w0_hbm.at[pl.ds(base, TPW)], wbuf0)
        pltpu.sync_copy(w1_hbm.at[pl.ds(base, TPW)], wbuf1)
        c1 = pltpu.async_copy(xbuf, xs_hbm.at[idx0], sem)
        c2 = pltpu.async_copy(xbuf, xs_hbm.at[idx1], sem)
        c3 = pltpu.async_copy(wbuf0, ws_hbm.at[idx0], sem)
        c4 = pltpu.async_copy(wbuf1, ws_hbm.at[idx1], sem)
        c1.wait()
        c2.wait()
        c3.wait()
        c4.wait()

    return scatter(x2, d0, d1, w0f, w1f)


def _combine(ys, d0, d1):
    mesh = plsc.VectorSubcoreMesh(core_axis_name="c", subcore_axis_name="s")
    HC = TPW // 2

    @functools.partial(
        pl.kernel, mesh=mesh,
        out_type=jax.ShapeDtypeStruct((S, D), jnp.float32),
        scratch_types=[
            pltpu.VMEM((HC, D), jnp.float32),
            pltpu.VMEM((HC, D), jnp.float32),
            pltpu.VMEM((HC,), jnp.int32),
            pltpu.VMEM((HC,), jnp.int32),
            pltpu.SemaphoreType.DMA,
        ],
    )
    def gather_add(ys_hbm, d0_hbm, d1_hbm, out_hbm, buf0, buf1, idx0, idx1,
                   sem):
        wid = lax.axis_index("s") * 2 + lax.axis_index("c")
        for half in range(2):
            base = wid * TPW + half * HC
            pltpu.sync_copy(d0_hbm.at[pl.ds(base, HC)], idx0)
            pltpu.sync_copy(d1_hbm.at[pl.ds(base, HC)], idx1)
            c1 = pltpu.async_copy(ys_hbm.at[idx0], buf0, sem)
            c2 = pltpu.async_copy(ys_hbm.at[idx1], buf1, sem)
            c1.wait()
            c2.wait()

            def row_add(r, carry):
                for cc in range(D // 16):
                    v = buf0[r, pl.ds(cc * 16, 16)] + buf1[r, pl.ds(cc * 16, 16)]
                    buf0[r, pl.ds(cc * 16, 16)] = v
                return carry

            lax.fori_loop(0, HC, row_add, 0)
            pltpu.sync_copy(buf0, out_hbm.at[pl.ds(base, HC)])

    return gather_add(ys, d0, d1)


@jax.jit
def kernel(x, gate_w, e_bias, Wg, Wu, Wd):
    b, s, d = x.shape
    x2 = x.reshape(s, d)
    gwp = jnp.zeros((LANES, D), jnp.float32).at[:E].set(gate_w)
    ebp = jnp.zeros((1, LANES), jnp.float32).at[0, :E].set(e_bias)

    d0s, d1s, w0f, w1f, eidr = _plan(x2, gwp, ebp)
    d0 = d0s.reshape(S)
    d1 = d1s.reshape(S)
    eid = eidr[0, :NT]

    idxs = jnp.arange(NT, dtype=jnp.int32)
    prev = jnp.concatenate([jnp.full((1,), -1, jnp.int32), eid[:-1]])
    first = (eid != prev).astype(jnp.int32)
    par = (jnp.cumsum(first) - 1) & 1
    fidx = jnp.where(first == 1, idxs, NT + 1)
    nxt = lax.cummin(fidx[::-1])[::-1]
    nxt = jnp.concatenate([nxt[1:], jnp.full((1,), NT + 1, jnp.int32)])
    pref = ((first == 1) & (nxt < NT)).astype(jnp.int32)
    nxte = eid[jnp.clip(nxt, 0, NT - 1)]
    meta = jnp.stack([par, first, nxte, pref, eid]).astype(jnp.int32)

    xs, wslot = _dispatch(x2, d0, d1, w0f, w1f)
    ys = _ffn(meta, xs, Wg, Wu, Wd, wslot)
    out = _combine(ys, d0, d1)
    return out.reshape(b, s, d)
